# baseline (speedup 1.0000x reference)
import functools

import jax
import jax.numpy as jnp
from jax import lax
from jax.experimental import pallas as pl
from jax.experimental.pallas import tpu as pltpu
from jax.experimental.pallas import tpu_sc as plsc

D_MODEL = 64

_info = plsc.get_sparse_core_info()
_NC = _info.num_cores
_NS = _info.num_subcores
_NW = _NC * _NS

_CHUNK = 1024


def _gather_kernel(table_hbm, idx_hbm, out_hbm, idx_v, rows_v, sem,
                   *, b_per_w, n_chunks):
    wid = lax.axis_index("s") * _NC + lax.axis_index("c")
    base = wid * b_per_w

    def step(i, carry):
        off = base + i * _CHUNK
        pltpu.sync_copy(idx_hbm.at[pl.ds(off, _CHUNK)], idx_v)
        pltpu.async_copy(table_hbm.at[idx_v], rows_v, sem).wait()
        pltpu.sync_copy(rows_v, out_hbm.at[pl.ds(off, _CHUNK)])
        return carry

    lax.fori_loop(0, n_chunks, step, 0, unroll=False)


def kernel(tokens, embeddings):
    orig_shape = tokens.shape
    idx = tokens.reshape(-1).astype(jnp.int32)
    b = idx.shape[0]
    b_per_w = b // _NW
    n_chunks = b_per_w // _CHUNK

    mesh = plsc.VectorSubcoreMesh(core_axis_name="c", subcore_axis_name="s")
    run = pl.kernel(
        functools.partial(_gather_kernel, b_per_w=b_per_w, n_chunks=n_chunks),
        mesh=mesh,
        out_type=jax.ShapeDtypeStruct((b, D_MODEL), jnp.float32),
        scratch_types=[
            pltpu.VMEM((_CHUNK,), jnp.int32),
            pltpu.VMEM((_CHUNK, D_MODEL), jnp.float32),
            pltpu.SemaphoreType.DMA,
        ],
        compiler_params=pltpu.CompilerParams(use_tc_tiling_on_sc=False),
    )
    out = run(embeddings, idx)
    return out.reshape(orig_shape + (D_MODEL,))


# speedup vs baseline: 1.0166x; 1.0166x over previous
import functools

import jax
import jax.numpy as jnp
from jax import lax
from jax.experimental import pallas as pl
from jax.experimental.pallas import tpu as pltpu
from jax.experimental.pallas import tpu_sc as plsc

D_MODEL = 64

_info = plsc.get_sparse_core_info()
_NC = _info.num_cores
_NS = _info.num_subcores
_NW = _NC * _NS

_CHUNK = 512
_NBUF = 2


def _gather_kernel(table_hbm, idx_hbm, out_hbm, idx_all, rows,
                   sem_g0, sem_g1, sem_o0, sem_o1, *, b_per_w, n_chunks):
    wid = lax.axis_index("s") * _NC + lax.axis_index("c")
    base = wid * b_per_w
    sems_g = [sem_g0, sem_g1]
    sems_o = [sem_o0, sem_o1]

    pltpu.sync_copy(idx_hbm.at[pl.ds(base, b_per_w)], idx_all)

    def gather_src(i):
        return table_hbm.at[idx_all.at[pl.ds(i * _CHUNK, _CHUNK)]]

    pltpu.async_copy(gather_src(0), rows.at[0], sems_g[0])
    pltpu.async_copy(gather_src(1), rows.at[1], sems_g[1])

    def group(g, carry):
        for b in range(_NBUF):
            i = g * _NBUF + b
            off = base + i * _CHUNK
            pltpu.make_async_copy(gather_src(i), rows.at[b], sems_g[b]).wait()
            out_cp = pltpu.make_async_copy(
                rows.at[b], out_hbm.at[pl.ds(off, _CHUNK)], sems_o[b])
            out_cp.start()
            out_cp.wait()
            nxt = i + _NBUF

            @pl.when(nxt < n_chunks)
            def _():
                pltpu.async_copy(gather_src(nxt), rows.at[b], sems_g[b])

        return carry

    lax.fori_loop(0, n_chunks // _NBUF, group, 0, unroll=False)


def kernel(tokens, embeddings):
    orig_shape = tokens.shape
    idx = tokens.reshape(-1).astype(jnp.int32)
    b = idx.shape[0]
    b_per_w = b // _NW
    n_chunks = b_per_w // _CHUNK

    mesh = plsc.VectorSubcoreMesh(core_axis_name="c", subcore_axis_name="s")
    run = pl.kernel(
        functools.partial(_gather_kernel, b_per_w=b_per_w, n_chunks=n_chunks),
        mesh=mesh,
        out_type=jax.ShapeDtypeStruct((b, D_MODEL), jnp.float32),
        scratch_types=[
            pltpu.VMEM((b_per_w,), jnp.int32),
            pltpu.VMEM((_NBUF, _CHUNK, D_MODEL), jnp.float32),
            pltpu.SemaphoreType.DMA,
            pltpu.SemaphoreType.DMA,
            pltpu.SemaphoreType.DMA,
            pltpu.SemaphoreType.DMA,
        ],
        compiler_params=pltpu.CompilerParams(use_tc_tiling_on_sc=False),
    )
    out = run(embeddings, idx)
    return out.reshape(orig_shape + (D_MODEL,))
